# baseline (speedup 1.0000x reference)
import functools

import jax
import jax.numpy as jnp
from jax import lax
from jax.experimental import pallas as pl
from jax.experimental.pallas import tpu as pltpu
from jax.experimental.pallas import tpu_sc as plsc

B = 16384
F = 26
V = 100000
D = 32
NN = 13

NW = 32
SAMPLES_PER_W = B // NW
CHUNK = 64
NCHUNK = SAMPLES_PER_W // CHUNK
IDX_PER_CHUNK = CHUNK * F
IDX_ROWS = IDX_PER_CHUNK // 128
TOTAL_IDX_ROWS = B * F // 128


def _sc_gather_reduce(gidx, emb_flat, lin_flat):
    mesh = plsc.VectorSubcoreMesh(core_axis_name="c", subcore_axis_name="s")

    @functools.partial(
        pl.kernel,
        mesh=mesh,
        compiler_params=pltpu.CompilerParams(use_tc_tiling_on_sc=False),
        out_type=[
            jax.ShapeDtypeStruct((B, D), jnp.float32),
            jax.ShapeDtypeStruct((B, D), jnp.float32),
            jax.ShapeDtypeStruct((B * F,), jnp.float32),
        ],
        scratch_types=[
            pltpu.VMEM((IDX_PER_CHUNK,), jnp.int32),
            pltpu.VMEM((IDX_PER_CHUNK, D), jnp.float32),
            pltpu.VMEM((IDX_PER_CHUNK,), jnp.float32),
            pltpu.VMEM((CHUNK, D), jnp.float32),
            pltpu.VMEM((CHUNK, D), jnp.float32),
            pltpu.SemaphoreType.DMA,
            pltpu.SemaphoreType.DMA,
        ],
    )
    def k(gidx_hbm, emb_hbm, lin_hbm, s_out, q_out, lg_out,
          idx_v, rows_v, lrows_v, sbuf, qbuf, sem_e, sem_l):
        w = lax.axis_index("s") * 2 + lax.axis_index("c")
        for kc in range(NCHUNK):
            base = (w * NCHUNK + kc) * IDX_PER_CHUNK
            pltpu.sync_copy(gidx_hbm.at[pl.ds(base, IDX_PER_CHUNK)], idx_v)
            for g in range(IDX_ROWS):
                pltpu.async_copy(emb_hbm.at[idx_v.at[pl.ds(g * 128, 128)]],
                                 rows_v.at[pl.ds(g * 128, 128)], sem_e).wait()
                pltpu.async_copy(lin_hbm.at[idx_v.at[pl.ds(g * 128, 128)]],
                                 lrows_v.at[pl.ds(g * 128, 128)], sem_l).wait()

            def c_body(c, _):
                def f_body(f, carry):
                    s0, s1, q0, q1 = carry
                    r = c * F + f
                    v0 = rows_v[r, pl.ds(0, 16)]
                    v1 = rows_v[r, pl.ds(16, 16)]
                    return (s0 + v0, s1 + v1, q0 + v0 * v0, q1 + v1 * v1)

                z = jnp.zeros((16,), jnp.float32)
                s0, s1, q0, q1 = lax.fori_loop(0, F, f_body, (z, z, z, z))
                sbuf[c, pl.ds(0, 16)] = s0
                sbuf[c, pl.ds(16, 16)] = s1
                qbuf[c, pl.ds(0, 16)] = q0
                qbuf[c, pl.ds(16, 16)] = q1
                return 0

            lax.fori_loop(0, CHUNK, c_body, 0)
            sample_base = w * SAMPLES_PER_W + kc * CHUNK
            pltpu.sync_copy(sbuf, s_out.at[pl.ds(sample_base, CHUNK)])
            pltpu.sync_copy(qbuf, q_out.at[pl.ds(sample_base, CHUNK)])
            pltpu.sync_copy(lrows_v, lg_out.at[pl.ds(base, IDX_PER_CHUNK)])

    return k(gidx, emb_flat, lin_flat)


def _combine_body(s_ref, q_ref, lg_ref, xn_ref, en_ref, wn_ref, bn_ref, o_ref):
    S = s_ref[...]
    Q = q_ref[...]
    lg = lg_ref[...]
    Xnb = xn_ref[...]
    en = en_ref[...]
    Sn = jnp.dot(Xnb, en, preferred_element_type=jnp.float32)
    Qn = jnp.dot(Xnb * Xnb, en * en, preferred_element_type=jnp.float32)
    Ssum = (S + Sn) * (1.0 / (F + NN))
    Qsum = (Q + Qn) * (1.0 / (F + NN))
    inter = 0.5 * (jnp.sum(Ssum * Ssum, axis=1, keepdims=True)
                   - jnp.sum(Qsum, axis=1, keepdims=True))
    linear_c = jnp.sum(lg, axis=1, keepdims=True) * (1.0 / F)
    linear_n = jnp.dot(Xnb, wn_ref[...].T, preferred_element_type=jnp.float32)
    o_ref[...] = linear_c + linear_n + bn_ref[0, 0] + inter


def _combine(S, Q, lg, Xn, emb_n, W_n, b_n):
    blk = 2048
    grid = (B // blk,)
    return pl.pallas_call(
        _combine_body,
        grid=grid,
        in_specs=[
            pl.BlockSpec((blk, D), lambda i: (i, 0)),
            pl.BlockSpec((blk, D), lambda i: (i, 0)),
            pl.BlockSpec((blk, F), lambda i: (i, 0)),
            pl.BlockSpec((blk, NN), lambda i: (i, 0)),
            pl.BlockSpec((NN, D), lambda i: (0, 0)),
            pl.BlockSpec((1, NN), lambda i: (0, 0)),
            pl.BlockSpec((1, 1), lambda i: (0, 0)),
        ],
        out_specs=pl.BlockSpec((blk, 1), lambda i: (i, 0)),
        out_shape=jax.ShapeDtypeStruct((B, 1), jnp.float32),
    )(S, Q, lg, Xn, emb_n, W_n, b_n)


def kernel(Xc, Xn, emb_c, lin_c, emb_n, W_n, b_n):
    gidx = (Xc.astype(jnp.int32)
            + (jnp.arange(F, dtype=jnp.int32) * V)[None, :])
    gidx = gidx.reshape(B * F)
    emb_flat = emb_c.reshape(F * V, D)
    lin_flat = lin_c.reshape(F * V)
    S, Q, lg = _sc_gather_reduce(gidx, emb_flat, lin_flat)
    lg = lg.reshape(B, F)
    return _combine(S, Q, lg, Xn, emb_n, W_n, b_n.reshape(1, 1))
